# Initial kernel scaffold; baseline (speedup 1.0000x reference)
import jax
import jax.numpy as jnp
from jax.experimental import pallas as pl


def kernel(x, edge_index, batch, edge_attr, W1, b1, W2, b2, Wout, bout):
    raise NotImplementedError("write your pallas kernel here")


# speedup vs baseline: 2.5629x; 2.5629x over previous
import functools

import jax
import jax.numpy as jnp
from jax import lax
from jax.experimental import pallas as pl
from jax.experimental.pallas import tpu as pltpu
from jax.experimental.pallas import tpu_sc as plsc

_N = 10000
_E = 320000
_DE = 16
_H = 128
_G = 64
_NC = 2
_NS = 16
_NW = _NC * _NS
_CH = 80
_CHE = 160
_EW = _CH * 128
_EP = _NW * _EW
_NP = 10112
_RPS = _NP // _NS
_BLK = 128
_GRID = _NP // _BLK


def _sc_mesh():
    return plsc.VectorSubcoreMesh(core_axis_name="c", subcore_axis_name="s")


def _agg_rows(table):
    @functools.partial(
        pl.kernel,
        mesh=_sc_mesh(),
        out_type=jax.ShapeDtypeStruct((_NC, _NP, _H), jnp.float32),
        scratch_types=[
            pltpu.VMEM((128,), jnp.int32),
            pltpu.VMEM((128,), jnp.int32),
            pltpu.VMEM((128,), jnp.int32),
            pltpu.VMEM((128,), jnp.int32),
            pltpu.VMEM((128, _H), jnp.float32),
            pltpu.VMEM((128, _H), jnp.float32),
            pltpu.VMEM_SHARED((_NP, _H), jnp.float32),
            pltpu.SemaphoreType.DMA,
            pltpu.SemaphoreType.DMA,
            pltpu.SemaphoreType.DMA,
            pltpu.SemaphoreType.DMA,
            pltpu.SemaphoreType.DMA,
            pltpu.SemaphoreType.DMA,
        ],
    )
    def k(table_h, src_h, dst_h, outX,
          s0, s1, d0, d1, g0, g1, accX, ss0, ss1, ds0, ds1, gs0, gs1):
        cid = lax.axis_index("c")
        sid = lax.axis_index("s")
        w = cid * _NS + sid
        base = sid * _RPS

        def zg(i, c):
            g0[i >> 3, pl.ds((i & 7) * 16, 16)] = jnp.zeros((16,), jnp.float32)
            return c
        lax.fori_loop(0, 128 * 8, zg, 0)
        for t in range(_RPS // 128):
            pltpu.sync_copy(g0, accX.at[pl.ds(base + t * 128, 128)])
        rem = _RPS % 128
        pltpu.sync_copy(g0.at[pl.ds(0, rem)], accX.at[pl.ds(base + _RPS - rem, rem)])
        plsc.subcore_barrier()

        sbuf = (s0, s1)
        dbuf = (d0, d1)
        gbuf = (g0, g1)
        ssem = (ss0, ss1)
        dsem = (ds0, ds1)
        gsem = (gs0, gs1)

        def s_copy(j, b):
            return pltpu.make_async_copy(src_h.at[w, j], sbuf[b], ssem[b])

        def d_copy(j, b):
            return pltpu.make_async_copy(dst_h.at[w, j], dbuf[b], dsem[b])

        def g_copy(j, b):
            return pltpu.make_async_copy(table_h.at[sbuf[b]], gbuf[b], gsem[b])

        s_copy(0, 0).start()
        d_copy(0, 0).start()
        s_copy(1, 1).start()
        d_copy(1, 1).start()
        s_copy(0, 0).wait()
        g_copy(0, 0).start()

        def body(i, c):
            for b in range(2):
                j = i * 2 + b
                q = 1 - b

                @pl.when(j + 1 < _CH)
                def _():
                    s_copy(j + 1, q).wait()
                    g_copy(j + 1, q).start()

                g_copy(j, b).wait()
                d_copy(j, b).wait()
                pltpu.sync_copy(gbuf[b], accX.at[dbuf[b]], add=True)

                @pl.when(j + 2 < _CH)
                def _():
                    s_copy(j + 2, b).start()
                    d_copy(j + 2, b).start()
            return c
        lax.fori_loop(0, _CH // 2, body, 0)

        plsc.subcore_barrier()
        pltpu.sync_copy(accX.at[pl.ds(base, _RPS)], outX.at[cid, pl.ds(base, _RPS)])

    return k


def _agg_edge():
    @functools.partial(
        pl.kernel,
        mesh=_sc_mesh(),
        out_type=jax.ShapeDtypeStruct((_NC, _NP, _H), jnp.float32),
        scratch_types=[
            pltpu.VMEM((64,), jnp.int32),
            pltpu.VMEM((64,), jnp.int32),
            pltpu.VMEM((64, 32), jnp.float32),
            pltpu.VMEM((64, 32), jnp.float32),
            pltpu.VMEM((64, _H), jnp.float32),
            pltpu.VMEM((64, _H), jnp.float32),
            pltpu.VMEM_SHARED((_NP, _H), jnp.float32),
            pltpu.SemaphoreType.DMA,
            pltpu.SemaphoreType.DMA,
            pltpu.SemaphoreType.DMA,
            pltpu.SemaphoreType.DMA,
        ],
    )
    def k(eaug_h, dst_h, out32, d0, d1, c0, c1, e0, e1, acc32, ds0, ds1, es0, es1):
        cid = lax.axis_index("c")
        sid = lax.axis_index("s")
        w = cid * _NS + sid
        base = sid * _RPS

        def ze(i, c):
            e0[i >> 3, pl.ds((i & 7) * 16, 16)] = jnp.zeros((16,), jnp.float32)
            e1[i >> 3, pl.ds((i & 7) * 16, 16)] = jnp.zeros((16,), jnp.float32)
            return c
        lax.fori_loop(0, 64 * 8, ze, 0)
        for t in range(_RPS // 64):
            pltpu.sync_copy(e0, acc32.at[pl.ds(base + t * 64, 64)])
        rem = _RPS % 64
        if rem:
            pltpu.sync_copy(e0.at[pl.ds(0, rem)], acc32.at[pl.ds(base + _RPS - rem, rem)])
        plsc.subcore_barrier()

        ebuf = (e0, e1)
        cbuf = (c0, c1)
        esem = (es0, es1)
        dbuf = (d0, d1)
        dsem = (ds0, ds1)

        def e_copy(j, b):
            return pltpu.make_async_copy(eaug_h.at[w, j], cbuf[b], esem[b])

        def d_copy(j, b):
            return pltpu.make_async_copy(dst_h.at[w, j], dbuf[b], dsem[b])

        e_copy(0, 0).start()
        d_copy(0, 0).start()

        def body(i, c):
            for b in range(2):
                j = i * 2 + b
                q = 1 - b

                @pl.when(j + 1 < _CHE)
                def _():
                    e_copy(j + 1, q).start()
                    d_copy(j + 1, q).start()

                e_copy(j, b).wait()
                d_copy(j, b).wait()

                def xp(t, cc):
                    ebuf[b][t >> 1, pl.ds((t & 1) * 16, 16)] = cbuf[b][t >> 1, pl.ds((t & 1) * 16, 16)]
                    return cc
                lax.fori_loop(0, 64 * 2, xp, 0)
                pltpu.sync_copy(ebuf[b], acc32.at[dbuf[b]], add=True)
            return c
        lax.fori_loop(0, _CHE // 2, body, 0)

        plsc.subcore_barrier()
        pltpu.sync_copy(acc32.at[pl.ds(base, _RPS)], out32.at[cid, pl.ds(base, _RPS)])

    return k


def _tc_layer(aggX, agg32, Wx, Waug):
    def body(ax_ref, a32_ref, wx_ref, waug_ref, o_ref):
        hx = ax_ref[0] + ax_ref[1]
        ha = a32_ref[0] + a32_ref[1]
        acc = jnp.dot(hx, wx_ref[...], preferred_element_type=jnp.float32)
        acc += jnp.dot(ha, waug_ref[...], preferred_element_type=jnp.float32)
        o_ref[...] = jnp.maximum(acc, 0.0)

    return pl.pallas_call(
        body,
        grid=(_GRID,),
        in_specs=[
            pl.BlockSpec((_NC, _BLK, _H), lambda i: (0, i, 0)),
            pl.BlockSpec((_NC, _BLK, _H), lambda i: (0, i, 0)),
            pl.BlockSpec((_H, _H), lambda i: (0, 0)),
            pl.BlockSpec((_H, _H), lambda i: (0, 0)),
        ],
        out_specs=pl.BlockSpec((_BLK, _H), lambda i: (i, 0)),
        out_shape=jax.ShapeDtypeStruct((_NP, _H), jnp.float32),
    )(aggX, agg32, Wx, Waug)


def _tc_layer_pool(aggX, agg32, Wx, Waug, batchI, Woutp, boutp):
    def body(ax_ref, a32_ref, wx_ref, waug_ref, b_ref, wo_ref, bo_ref,
             o_ref, pool_acc, cnt_acc):
        i = pl.program_id(0)

        @pl.when(i == 0)
        def _():
            pool_acc[...] = jnp.zeros_like(pool_acc)
            cnt_acc[...] = jnp.zeros_like(cnt_acc)

        hx = ax_ref[0] + ax_ref[1]
        ha = a32_ref[0] + a32_ref[1]
        acc = jnp.dot(hx, wx_ref[...], preferred_element_type=jnp.float32)
        acc += jnp.dot(ha, waug_ref[...], preferred_element_type=jnp.float32)
        h2 = jnp.maximum(acc, 0.0)

        oh = (lax.broadcasted_iota(jnp.int32, (_G, _BLK), 0) == b_ref[0]).astype(jnp.float32)
        pool_acc[...] += jnp.dot(oh, h2, preferred_element_type=jnp.float32)
        cnt_acc[...] += jnp.broadcast_to(jnp.sum(oh, axis=1, keepdims=True), (_G, _BLK))

        @pl.when(i == _GRID - 1)
        def _():
            pooled = pool_acc[...] / jnp.maximum(cnt_acc[...], 1.0)
            o_ref[...] = (jnp.dot(pooled, wo_ref[...], preferred_element_type=jnp.float32)
                          + bo_ref[...])

    return pl.pallas_call(
        body,
        grid=(_GRID,),
        in_specs=[
            pl.BlockSpec((_NC, _BLK, _H), lambda i: (0, i, 0)),
            pl.BlockSpec((_NC, _BLK, _H), lambda i: (0, i, 0)),
            pl.BlockSpec((_H, _H), lambda i: (0, 0)),
            pl.BlockSpec((_H, _H), lambda i: (0, 0)),
            pl.BlockSpec((1, 1, _BLK), lambda i: (i, 0, 0)),
            pl.BlockSpec((_H, _H), lambda i: (0, 0)),
            pl.BlockSpec((1, _H), lambda i: (0, 0)),
        ],
        out_specs=pl.BlockSpec((_G, _H), lambda i: (0, 0)),
        out_shape=jax.ShapeDtypeStruct((_G, _H), jnp.float32),
        scratch_shapes=[
            pltpu.VMEM((_G, _H), jnp.float32),
            pltpu.VMEM((_G, _BLK), jnp.float32),
        ],
    )(aggX, agg32, Wx, Waug, batchI, Woutp, boutp)


def _agg_first(table, srcI, dstI, eaugI, dstE):
    aggX = _agg_rows(table)(table, srcI, dstI)
    agg32 = _agg_edge()(eaugI, dstE)
    return aggX, agg32


def _agg_second(table, srcI, dstI):
    return _agg_rows(table)(table, srcI, dstI)


def kernel(x, edge_index, batch, edge_attr, W1, b1, W2, b2, Wout, bout):
    src = edge_index[0]
    dst = edge_index[1]
    srcI = jnp.pad(src, (0, _EP - _E)).reshape(_NW, _CH, 128)
    dstI = jnp.pad(dst, (0, _EP - _E), constant_values=_N).reshape(_NW, _CH, 128)

    dstE = jnp.pad(dst, (0, _EP - _E), constant_values=_N).reshape(_NW, _CHE, 64)
    eaug = jnp.concatenate([edge_attr, jnp.ones((_E, 1), jnp.float32)], axis=1)
    eaug = jnp.pad(eaug, ((0, _EP - _E), (0, 32 - (_DE + 1))))
    eaugI = eaug.reshape(_NW, _CHE, 64, 32)

    W1x = W1[:_H]
    W1aug = jnp.concatenate([W1[_H:], b1[None, :], jnp.zeros((_H - _DE - 1, _H), jnp.float32)])
    W2x = W2[:_H]
    W2aug = jnp.concatenate([W2[_H:], b2[None, :], jnp.zeros((_H - _DE - 1, _H), jnp.float32)])
    Woutp = jnp.pad(Wout, ((0, 0), (0, _H - Wout.shape[1])))
    boutp = jnp.pad(bout[None, :], ((0, 0), (0, _H - bout.shape[0])))
    batchI = jnp.concatenate([batch, jnp.full((_NP - _N,), _G, jnp.int32)]).reshape(_GRID, 1, _BLK)

    aggX, agg32 = _agg_first(x, srcI, dstI, eaugI, dstE)
    h1 = _tc_layer(aggX, agg32, W1x, W1aug)
    aggH = _agg_second(h1, srcI, dstI)
    out = _tc_layer_pool(aggH, agg32, W2x, W2aug, batchI, Woutp, boutp)
    return out[:, : Wout.shape[1]]
